# Initial kernel scaffold; baseline (speedup 1.0000x reference)
import jax
import jax.numpy as jnp
from jax.experimental import pallas as pl


def kernel(x, edge_index, W1, b1, W2, b2):
    raise NotImplementedError("write your pallas kernel here")


# speedup vs baseline: 9.8595x; 9.8595x over previous
import functools

import jax
import jax.numpy as jnp
from jax import lax
from jax.experimental import pallas as pl
from jax.experimental.pallas import tpu as pltpu
from jax.experimental.pallas import tpu_sc as plsc

N = 10000
NP = 10240
D = 128
E = 320000
NC = 2
NS = 16
NW = NC * NS
EPW = E // NW
CH = 128
NCH = 80
DUMP = N
ZR = NP // NS
RB = 1024
GRID = NP // RB

_mesh = plsc.VectorSubcoreMesh(core_axis_name="c", subcore_axis_name="s")


@functools.partial(
    pl.kernel,
    out_type=jax.ShapeDtypeStruct((NC, NP), jnp.float32),
    mesh=_mesh,
    scratch_types=[
        pltpu.VMEM((NCH, CH), jnp.int32),
        pltpu.VMEM((CH,), jnp.float32),
        pltpu.VMEM((ZR,), jnp.float32),
        pltpu.VMEM_SHARED((NP,), jnp.float32),
    ],
)
def _deg_kernel(dst_hbm, out_hbm, dst_v, ones_v, z_v, deg_sh):
    cid = lax.axis_index("c")
    sid = lax.axis_index("s")
    w = cid * NS + sid
    pltpu.sync_copy(dst_hbm.at[w], dst_v)
    one = jnp.full((16,), 1.0, dtype=jnp.float32)
    zero = jnp.zeros((16,), dtype=jnp.float32)
    for i in range(CH // 16):
        ones_v[pl.ds(i * 16, 16)] = one
    for i in range(ZR // 16):
        z_v[pl.ds(i * 16, 16)] = zero
    pltpu.sync_copy(z_v, deg_sh.at[pl.ds(sid * ZR, ZR)])
    plsc.subcore_barrier()

    @pl.loop(0, NCH)
    def _(c):
        pltpu.sync_copy(ones_v, deg_sh.at[dst_v.at[c]], add=True)

    plsc.subcore_barrier()

    @pl.when(sid == 0)
    def _():
        pltpu.sync_copy(deg_sh, out_hbm.at[cid])


@functools.partial(
    pl.kernel,
    out_type=jax.ShapeDtypeStruct((NC, NP, D), jnp.float32),
    mesh=_mesh,
    scratch_types=[
        pltpu.VMEM((NCH, CH), jnp.int32),
        pltpu.VMEM((NCH, CH), jnp.int32),
        pltpu.VMEM((CH, D), jnp.float32),
        pltpu.VMEM_SHARED((NP, D), jnp.float32),
        pltpu.SemaphoreType.DMA,
    ],
)
def _agg_kernel(hs_hbm, src_hbm, dst_hbm, zero_hbm, out_hbm,
                src_v, dst_v, buf, agg_sh, sem):
    cid = lax.axis_index("c")
    sid = lax.axis_index("s")
    w = cid * NS + sid
    pltpu.sync_copy(src_hbm.at[w], src_v)
    pltpu.sync_copy(dst_hbm.at[w], dst_v)
    rows = pl.ds(sid * ZR, ZR)

    @pl.when(cid == 0)
    def _():
        pltpu.sync_copy(hs_hbm.at[rows], agg_sh.at[rows])

    @pl.when(cid == 1)
    def _():
        pltpu.sync_copy(zero_hbm, agg_sh.at[rows])

    plsc.subcore_barrier()

    @pl.loop(0, NCH)
    def _(c):
        pltpu.async_copy(hs_hbm.at[src_v.at[c]], buf, sem).wait()
        pltpu.sync_copy(buf, agg_sh.at[dst_v.at[c]], add=True)

    plsc.subcore_barrier()
    pltpu.sync_copy(agg_sh.at[rows], out_hbm.at[cid, rows])


def _tc1_body(x_ref, w_ref, p_ref, hs_ref, dis_ref):
    p = p_ref[...]
    deg = p[0] + p[1] + 1.0
    dis = lax.rsqrt(deg)
    h = jnp.dot(x_ref[...], w_ref[...], preferred_element_type=jnp.float32)
    hs_ref[...] = h * dis
    dis_ref[...] = dis


_tc1 = pl.pallas_call(
    _tc1_body,
    grid=(GRID,),
    in_specs=[
        pl.BlockSpec((RB, D), lambda i: (i, 0)),
        pl.BlockSpec((D, D), lambda i: (0, 0)),
        pl.BlockSpec((NC, RB, 1), lambda i: (0, i, 0)),
    ],
    out_specs=[
        pl.BlockSpec((RB, D), lambda i: (i, 0)),
        pl.BlockSpec((RB, 1), lambda i: (i, 0)),
    ],
    out_shape=[
        jax.ShapeDtypeStruct((NP, D), jnp.float32),
        jax.ShapeDtypeStruct((NP, 1), jnp.float32),
    ],
)


def _tc2_body(a_ref, dis_ref, b_ref, w_ref, hs2_ref):
    a = a_ref[...]
    dis = dis_ref[...]
    out1 = (a[0] + a[1]) * dis + b_ref[...]
    h1 = jnp.maximum(out1, 0.0)
    h2 = jnp.dot(h1, w_ref[...], preferred_element_type=jnp.float32)
    hs2_ref[...] = h2 * dis


_tc2 = pl.pallas_call(
    _tc2_body,
    grid=(GRID,),
    in_specs=[
        pl.BlockSpec((NC, RB, D), lambda i: (0, i, 0)),
        pl.BlockSpec((RB, 1), lambda i: (i, 0)),
        pl.BlockSpec((1, D), lambda i: (0, 0)),
        pl.BlockSpec((D, D), lambda i: (0, 0)),
    ],
    out_specs=pl.BlockSpec((RB, D), lambda i: (i, 0)),
    out_shape=jax.ShapeDtypeStruct((NP, D), jnp.float32),
)


def _tc3_body(a_ref, dis_ref, b_ref, o_ref):
    a = a_ref[...]
    out = (a[0] + a[1]) * dis_ref[...] + b_ref[...]
    m = jnp.max(out, axis=1, keepdims=True)
    ex = jnp.exp(out - m)
    s = jnp.sum(ex, axis=1, keepdims=True)
    o_ref[...] = (out - m) - jnp.log(s)


_tc3 = pl.pallas_call(
    _tc3_body,
    grid=(GRID,),
    in_specs=[
        pl.BlockSpec((NC, RB, D), lambda i: (0, i, 0)),
        pl.BlockSpec((RB, 1), lambda i: (i, 0)),
        pl.BlockSpec((1, D), lambda i: (0, 0)),
    ],
    out_specs=pl.BlockSpec((RB, D), lambda i: (i, 0)),
    out_shape=jax.ShapeDtypeStruct((NP, D), jnp.float32),
)


def kernel(x, edge_index, W1, b1, W2, b2):
    ei = edge_index.astype(jnp.int32)
    srcp = jnp.pad(ei[0].reshape(NW, EPW),
                   ((0, 0), (0, NCH * CH - EPW))).reshape(NW, NCH, CH)
    dstp = jnp.pad(ei[1].reshape(NW, EPW),
                   ((0, 0), (0, NCH * CH - EPW)),
                   constant_values=DUMP).reshape(NW, NCH, CH)
    xp = jnp.pad(x, ((0, NP - N), (0, 0)))
    zero = jnp.zeros((ZR, D), jnp.float32)

    degp = _deg_kernel(dstp)
    hs1, dis = _tc1(xp, W1, degp[:, :, None])
    a1 = _agg_kernel(hs1, srcp, dstp, zero)
    hs2 = _tc2(a1, dis, b1.reshape(1, D), W2)
    a2 = _agg_kernel(hs2, srcp, dstp, zero)
    out = _tc3(a2, dis, b2.reshape(1, D))
    return out[:N]


# speedup vs baseline: 10.5180x; 1.0668x over previous
import functools

import jax
import jax.numpy as jnp
from jax import lax
from jax.experimental import pallas as pl
from jax.experimental.pallas import tpu as pltpu
from jax.experimental.pallas import tpu_sc as plsc

N = 10000
NP = 10240
D = 128
E = 320000
NC = 2
NS = 16
NW = NC * NS
EPW = E // NW
CH = 128
NCH = 80
DUMP = N
ZR = NP // NS
RB = 1024
GRID = NP // RB

_mesh = plsc.VectorSubcoreMesh(core_axis_name="c", subcore_axis_name="s")


@functools.partial(
    pl.kernel,
    out_type=jax.ShapeDtypeStruct((NC, NP), jnp.float32),
    mesh=_mesh,
    scratch_types=[
        pltpu.VMEM((NCH, CH), jnp.int32),
        pltpu.VMEM((CH,), jnp.float32),
        pltpu.VMEM((ZR,), jnp.float32),
        pltpu.VMEM_SHARED((NP,), jnp.float32),
    ],
)
def _deg_kernel(dst_hbm, out_hbm, dst_v, ones_v, z_v, deg_sh):
    cid = lax.axis_index("c")
    sid = lax.axis_index("s")
    w = cid * NS + sid
    pltpu.sync_copy(dst_hbm.at[w], dst_v)
    one = jnp.full((16,), 1.0, dtype=jnp.float32)
    zero = jnp.zeros((16,), dtype=jnp.float32)
    for i in range(CH // 16):
        ones_v[pl.ds(i * 16, 16)] = one
    for i in range(ZR // 16):
        z_v[pl.ds(i * 16, 16)] = zero
    pltpu.sync_copy(z_v, deg_sh.at[pl.ds(sid * ZR, ZR)])
    plsc.subcore_barrier()

    @pl.loop(0, NCH)
    def _(c):
        pltpu.sync_copy(ones_v, deg_sh.at[dst_v.at[c]], add=True)

    plsc.subcore_barrier()

    @pl.when(sid == 0)
    def _():
        pltpu.sync_copy(deg_sh, out_hbm.at[cid])


@functools.partial(
    pl.kernel,
    out_type=jax.ShapeDtypeStruct((NC, NP, D), jnp.float32),
    mesh=_mesh,
    scratch_types=[
        pltpu.VMEM((1, CH), jnp.int32),
        pltpu.VMEM((1, CH), jnp.int32),
        pltpu.VMEM((NCH, CH), jnp.int32),
        pltpu.VMEM((CH, D), jnp.float32),
        pltpu.VMEM((CH, D), jnp.float32),
        pltpu.VMEM_SHARED((NP, D), jnp.float32),
        pltpu.SemaphoreType.DMA,
        pltpu.SemaphoreType.DMA,
        pltpu.SemaphoreType.DMA,
        pltpu.SemaphoreType.DMA,
        pltpu.SemaphoreType.DMA,
        pltpu.SemaphoreType.DMA,
    ],
)
def _agg_kernel(hs_hbm, src_hbm, dst_hbm, zero_hbm, out_hbm,
                si0, si1, dst_v, buf0, buf1, agg_sh,
                ssem0, ssem1, gsem0, gsem1, wsem0, wsem1):
    cid = lax.axis_index("c")
    sid = lax.axis_index("s")
    w = cid * NS + sid
    pltpu.sync_copy(dst_hbm.at[w], dst_v)
    rows = pl.ds(sid * ZR, ZR)

    @pl.when(cid == 0)
    def _():
        pltpu.sync_copy(hs_hbm.at[rows], agg_sh.at[rows])

    @pl.when(cid == 1)
    def _():
        pltpu.sync_copy(zero_hbm, agg_sh.at[rows])

    plsc.subcore_barrier()

    pltpu.async_copy(src_hbm.at[w, pl.ds(0, 1)], si0, ssem0)
    pltpu.async_copy(src_hbm.at[w, pl.ds(1, 1)], si1, ssem1)
    pltpu.make_async_copy(src_hbm.at[w, pl.ds(0, 1)], si0, ssem0).wait()
    pltpu.async_copy(hs_hbm.at[si0.at[0]], buf0, gsem0)
    pltpu.make_async_copy(src_hbm.at[w, pl.ds(1, 1)], si1, ssem1).wait()
    pltpu.async_copy(hs_hbm.at[si1.at[0]], buf1, gsem1)

    @pl.loop(0, NCH, step=2)
    def _(c):
        pltpu.make_async_copy(hs_hbm.at[si0.at[0]], buf0, gsem0).wait()
        pltpu.async_copy(src_hbm.at[w, pl.ds(c + 2, 1)], si0, ssem0)
        pltpu.async_copy(buf0, agg_sh.at[dst_v.at[c]], wsem0, add=True)

        pltpu.make_async_copy(hs_hbm.at[si1.at[0]], buf1, gsem1).wait()
        pltpu.async_copy(src_hbm.at[w, pl.ds(c + 3, 1)], si1, ssem1)
        pltpu.async_copy(buf1, agg_sh.at[dst_v.at[c + 1]], wsem1, add=True)

        pltpu.make_async_copy(buf0, agg_sh.at[dst_v.at[c]], wsem0).wait()
        pltpu.make_async_copy(src_hbm.at[w, pl.ds(c + 2, 1)], si0, ssem0).wait()

        @pl.when(c + 2 < NCH)
        def _():
            pltpu.async_copy(hs_hbm.at[si0.at[0]], buf0, gsem0)

        pltpu.make_async_copy(buf1, agg_sh.at[dst_v.at[c + 1]], wsem1).wait()
        pltpu.make_async_copy(src_hbm.at[w, pl.ds(c + 3, 1)], si1, ssem1).wait()

        @pl.when(c + 3 < NCH)
        def _():
            pltpu.async_copy(hs_hbm.at[si1.at[0]], buf1, gsem1)

    plsc.subcore_barrier()
    pltpu.sync_copy(agg_sh.at[rows], out_hbm.at[cid, rows])


def _tc1_body(x_ref, w_ref, p_ref, hs_ref, dis_ref):
    p = p_ref[...]
    deg = p[0] + p[1] + 1.0
    dis = lax.rsqrt(deg)
    h = jnp.dot(x_ref[...], w_ref[...], preferred_element_type=jnp.float32)
    hs_ref[...] = h * dis
    dis_ref[...] = dis


_tc1 = pl.pallas_call(
    _tc1_body,
    grid=(GRID,),
    in_specs=[
        pl.BlockSpec((RB, D), lambda i: (i, 0)),
        pl.BlockSpec((D, D), lambda i: (0, 0)),
        pl.BlockSpec((NC, RB, 1), lambda i: (0, i, 0)),
    ],
    out_specs=[
        pl.BlockSpec((RB, D), lambda i: (i, 0)),
        pl.BlockSpec((RB, 1), lambda i: (i, 0)),
    ],
    out_shape=[
        jax.ShapeDtypeStruct((NP, D), jnp.float32),
        jax.ShapeDtypeStruct((NP, 1), jnp.float32),
    ],
)


def _tc2_body(a_ref, dis_ref, b_ref, w_ref, hs2_ref):
    a = a_ref[...]
    dis = dis_ref[...]
    out1 = (a[0] + a[1]) * dis + b_ref[...]
    h1 = jnp.maximum(out1, 0.0)
    h2 = jnp.dot(h1, w_ref[...], preferred_element_type=jnp.float32)
    hs2_ref[...] = h2 * dis


_tc2 = pl.pallas_call(
    _tc2_body,
    grid=(GRID,),
    in_specs=[
        pl.BlockSpec((NC, RB, D), lambda i: (0, i, 0)),
        pl.BlockSpec((RB, 1), lambda i: (i, 0)),
        pl.BlockSpec((1, D), lambda i: (0, 0)),
        pl.BlockSpec((D, D), lambda i: (0, 0)),
    ],
    out_specs=pl.BlockSpec((RB, D), lambda i: (i, 0)),
    out_shape=jax.ShapeDtypeStruct((NP, D), jnp.float32),
)


def _tc3_body(a_ref, dis_ref, b_ref, o_ref):
    a = a_ref[...]
    out = (a[0] + a[1]) * dis_ref[...] + b_ref[...]
    m = jnp.max(out, axis=1, keepdims=True)
    ex = jnp.exp(out - m)
    s = jnp.sum(ex, axis=1, keepdims=True)
    o_ref[...] = (out - m) - jnp.log(s)


_tc3 = pl.pallas_call(
    _tc3_body,
    grid=(GRID,),
    in_specs=[
        pl.BlockSpec((NC, RB, D), lambda i: (0, i, 0)),
        pl.BlockSpec((RB, 1), lambda i: (i, 0)),
        pl.BlockSpec((1, D), lambda i: (0, 0)),
    ],
    out_specs=pl.BlockSpec((RB, D), lambda i: (i, 0)),
    out_shape=jax.ShapeDtypeStruct((NP, D), jnp.float32),
)


def kernel(x, edge_index, W1, b1, W2, b2):
    ei = edge_index.astype(jnp.int32)
    srcp = jnp.pad(ei[0].reshape(NW, EPW),
                   ((0, 0), (0, (NCH + 2) * CH - EPW))).reshape(NW, NCH + 2, CH)
    dstp = jnp.pad(ei[1].reshape(NW, EPW),
                   ((0, 0), (0, NCH * CH - EPW)),
                   constant_values=DUMP).reshape(NW, NCH, CH)
    xp = jnp.pad(x, ((0, NP - N), (0, 0)))
    zero = jnp.zeros((ZR, D), jnp.float32)

    degp = _deg_kernel(dstp)
    hs1, dis = _tc1(xp, W1, degp[:, :, None])
    a1 = _agg_kernel(hs1, srcp, dstp, zero)
    hs2 = _tc2(a1, dis, b1.reshape(1, D), W2)
    a2 = _agg_kernel(hs2, srcp, dstp, zero)
    out = _tc3(a2, dis, b2.reshape(1, D))
    return out[:N]
